# baseline (speedup 1.0000x reference)
import functools

import jax
import jax.numpy as jnp
from jax import lax
from jax.experimental import pallas as pl
from jax.experimental.pallas import tpu as pltpu
from jax.experimental.pallas import tpu_sc as plsc

BATCH = 64
HIDDEN = 4096
VOCAB = 32000


_N_WORKERS = 8
_ROWS_PER_W = BATCH // _N_WORKERS


@functools.cache
def _make_sc_gather():
    mesh = plsc.VectorSubcoreMesh(core_axis_name="c", subcore_axis_name="s")

    @functools.partial(
        pl.kernel,
        mesh=mesh,
        out_type=jax.ShapeDtypeStruct((BATCH, HIDDEN), jnp.float32),
        scratch_types=[
            pltpu.VMEM((_ROWS_PER_W,), jnp.int32),
            pltpu.VMEM((_ROWS_PER_W, HIDDEN), jnp.float32),
            pltpu.SemaphoreType.DMA,
        ],
    )
    def _sc_gather(idx_hbm, table_hbm, out_hbm, idx_v, rows_v, sem):
        wid = lax.axis_index("s") * 2 + lax.axis_index("c")

        @pl.when(wid < _N_WORKERS)
        def _():
            base = wid * _ROWS_PER_W
            pltpu.sync_copy(idx_hbm.at[pl.ds(base, _ROWS_PER_W)], idx_v)
            pltpu.async_copy(table_hbm.at[idx_v], rows_v, sem).wait()
            pltpu.sync_copy(rows_v, out_hbm.at[pl.ds(base, _ROWS_PER_W)])

    return _sc_gather


_VB = 640


def _proj_body(ids_ref, table_ref, w_ref, b_ref, out_ref, emb_vmem, sem):
    @pl.when(pl.program_id(0) == 0)
    def _():
        for i in range(BATCH):
            pltpu.make_async_copy(
                table_ref.at[pl.ds(ids_ref[i], 1)],
                emb_vmem.at[pl.ds(i, 1)],
                sem,
            ).start()
        for i in range(BATCH):
            pltpu.make_async_copy(
                table_ref.at[pl.ds(ids_ref[i], 1)],
                emb_vmem.at[pl.ds(i, 1)],
                sem,
            ).wait()

    acc = lax.dot_general(
        emb_vmem[...].astype(jnp.bfloat16),
        w_ref[...].astype(jnp.bfloat16),
        dimension_numbers=(((1,), (1,)), ((), ())),
        preferred_element_type=jnp.float32,
    )
    out_ref[:, 0, :] = acc + b_ref[...]


def _projection(ids, table, w_out, b_out):
    return pl.pallas_call(
        _proj_body,
        grid=(VOCAB // _VB,),
        in_specs=[
            pl.BlockSpec(memory_space=pltpu.SMEM),
            pl.BlockSpec(memory_space=pltpu.MemorySpace.HBM),
            pl.BlockSpec((_VB, HIDDEN), lambda j: (j, 0)),
            pl.BlockSpec((1, _VB), lambda j: (0, j)),
        ],
        out_specs=pl.BlockSpec((BATCH, 1, _VB), lambda j: (0, 0, j)),
        out_shape=jax.ShapeDtypeStruct((BATCH, 1, VOCAB), jnp.float32),
        scratch_shapes=[
            pltpu.VMEM((BATCH, HIDDEN), jnp.float32),
            pltpu.SemaphoreType.DMA,
        ],
        compiler_params=pltpu.CompilerParams(
            dimension_semantics=("arbitrary",),
        ),
    )(ids, table, w_out, b_out)


def kernel(input_ids, embed_table, W_out, b_out):
    ids = input_ids.reshape(BATCH).astype(jnp.int32)
    emb = _make_sc_gather()(ids, embed_table)
    logits = _projection(ids, embed_table, W_out, b_out.reshape(1, VOCAB))
    return (logits, emb.reshape(BATCH, 1, HIDDEN))


# speedup vs baseline: 1.0101x; 1.0101x over previous
import functools

import jax
import jax.numpy as jnp
from jax import lax
from jax.experimental import pallas as pl
from jax.experimental.pallas import tpu as pltpu
from jax.experimental.pallas import tpu_sc as plsc

BATCH = 64
HIDDEN = 4096
VOCAB = 32000


_N_WORKERS = 8
_ROWS_PER_W = BATCH // _N_WORKERS


@functools.cache
def _make_sc_gather():
    mesh = plsc.ScalarSubcoreMesh(axis_name="c", num_cores=1)

    @functools.partial(
        pl.kernel,
        mesh=mesh,
        out_type=jax.ShapeDtypeStruct((BATCH, HIDDEN), jnp.float32),
        scratch_types=[
            pltpu.SMEM((BATCH,), jnp.int32),
            pltpu.SemaphoreType.DMA,
        ],
    )
    def _sc_gather(idx_hbm, table_hbm, out_hbm, idx_s, sem):
        pltpu.sync_copy(idx_hbm, idx_s)
        for i in range(BATCH):
            pltpu.make_async_copy(
                table_hbm.at[pl.ds(idx_s[i], 1)],
                out_hbm.at[pl.ds(i, 1)],
                sem,
            ).start()
        for i in range(BATCH):
            pltpu.make_async_copy(
                table_hbm.at[pl.ds(idx_s[i], 1)],
                out_hbm.at[pl.ds(i, 1)],
                sem,
            ).wait()

    return _sc_gather


_VB = 640


def _proj_body(ids_ref, table_ref, w_ref, b_ref, out_ref, emb_vmem, sem):
    @pl.when(pl.program_id(0) == 0)
    def _():
        for i in range(BATCH):
            pltpu.make_async_copy(
                table_ref.at[pl.ds(ids_ref[i], 1)],
                emb_vmem.at[pl.ds(i, 1)],
                sem,
            ).start()
        for i in range(BATCH):
            pltpu.make_async_copy(
                table_ref.at[pl.ds(ids_ref[i], 1)],
                emb_vmem.at[pl.ds(i, 1)],
                sem,
            ).wait()

    acc = lax.dot_general(
        emb_vmem[...].astype(jnp.bfloat16),
        w_ref[...].astype(jnp.bfloat16),
        dimension_numbers=(((1,), (1,)), ((), ())),
        preferred_element_type=jnp.float32,
    )
    out_ref[:, 0, :] = acc + b_ref[...]


def _projection(ids, table, w_out, b_out):
    return pl.pallas_call(
        _proj_body,
        grid=(VOCAB // _VB,),
        in_specs=[
            pl.BlockSpec(memory_space=pltpu.SMEM),
            pl.BlockSpec(memory_space=pltpu.MemorySpace.HBM),
            pl.BlockSpec((_VB, HIDDEN), lambda j: (j, 0)),
            pl.BlockSpec((1, _VB), lambda j: (0, j)),
        ],
        out_specs=pl.BlockSpec((BATCH, 1, _VB), lambda j: (0, 0, j)),
        out_shape=jax.ShapeDtypeStruct((BATCH, 1, VOCAB), jnp.float32),
        scratch_shapes=[
            pltpu.VMEM((BATCH, HIDDEN), jnp.float32),
            pltpu.SemaphoreType.DMA,
        ],
        compiler_params=pltpu.CompilerParams(
            dimension_semantics=("arbitrary",),
        ),
    )(ids, table, w_out, b_out)


def kernel(input_ids, embed_table, W_out, b_out):
    ids = input_ids.reshape(BATCH).astype(jnp.int32)
    emb = _make_sc_gather()(ids, embed_table)
    logits = _projection(ids, embed_table, W_out, b_out.reshape(1, VOCAB))
    return (logits, emb.reshape(BATCH, 1, HIDDEN))
